# Initial kernel scaffold; baseline (speedup 1.0000x reference)
import jax
import jax.numpy as jnp
from jax.experimental import pallas as pl


def kernel(x, e, W1, a1s, a1d, b1, W2, a2s, a2d, b2, W3, a3s, a3d, b3, We, be, W9, b9, edge_index):
    raise NotImplementedError("write your pallas kernel here")


# speedup vs baseline: 41.3476x; 41.3476x over previous
import functools

import jax
import jax.numpy as jnp
from jax import lax
from jax.experimental import pallas as pl
from jax.experimental.pallas import tpu as pltpu
from jax.experimental.pallas import tpu_sc as plsc

N = 10000
E = 320000
DPAD = 16
NW = 32
NN = 10240
NPT = NN // 16

E2 = E + N
EPT2 = 10368
NCH2 = EPT2 // 128
EPAD2 = NW * EPT2

EPT = 10112
NCH = EPT // 128
EPAD = NW * EPT

_f32 = jnp.float32


def _prep_from_x(x_ref, w_ref, as_ref, ad_ref, hp_ref, hs_ref, hd_ref, c_ref, *, d_out):
    h = jnp.dot(x_ref[...], w_ref[...], preferred_element_type=_f32)
    col = lax.broadcasted_iota(jnp.int32, (N, DPAD), 1)
    hp_ref[...] = h + jnp.where(col == d_out, 1.0, 0.0).astype(_f32)
    hs = jnp.dot(h, as_ref[...], preferred_element_type=_f32)
    hd = jnp.dot(h, ad_ref[...], preferred_element_type=_f32)
    hs_ref[...] = hs
    hd_ref[...] = hd
    stab = jnp.max(hs) + hd
    c_ref[...] = jnp.maximum(stab, 0.2 * stab)


def _prep_from_acc(acc_ref, b_ref, w_ref, as_ref, ad_ref, hp_ref, hs_ref, hd_ref,
                   c_ref, *, d_prev, d_out):
    num = acc_ref[:N, :] + acc_ref[N:, :]
    den = num[:, d_prev:d_prev + 1] + 1e-16
    hprev = jnp.maximum(num / den + b_ref[...], 0.0)
    h = jnp.dot(hprev, w_ref[...], preferred_element_type=_f32)
    col = lax.broadcasted_iota(jnp.int32, (N, DPAD), 1)
    hp_ref[...] = h + jnp.where(col == d_out, 1.0, 0.0).astype(_f32)
    hs = jnp.dot(h, as_ref[...], preferred_element_type=_f32)
    hd = jnp.dot(h, ad_ref[...], preferred_element_type=_f32)
    hs_ref[...] = hs
    hd_ref[...] = hd
    stab = jnp.max(hs) + hd
    c_ref[...] = jnp.maximum(stab, 0.2 * stab)


def _prep_final(acc_ref, b_ref, wa_ref, wb_ref, be_ref, p_ref, q_ref, *, d_prev):
    num = acc_ref[:N, :] + acc_ref[N:, :]
    den = num[:, d_prev:d_prev + 1] + 1e-16
    h = jnp.maximum(num / den + b_ref[...], 0.0)
    p_ref[...] = jnp.dot(h, wa_ref[...], preferred_element_type=_f32) + be_ref[...]
    q_ref[...] = jnp.dot(h, wb_ref[...], preferred_element_type=_f32)


def _tc_prep_x(x, wp, asp, adp, d_out):
    return pl.pallas_call(
        functools.partial(_prep_from_x, d_out=d_out),
        out_shape=[
            jax.ShapeDtypeStruct((N, DPAD), _f32),
            jax.ShapeDtypeStruct((N, 1), _f32),
            jax.ShapeDtypeStruct((N, 1), _f32),
            jax.ShapeDtypeStruct((N, 1), _f32),
        ],
    )(x, wp, asp, adp)


def _tc_prep_acc(acc2, bp, wp, asp, adp, d_prev, d_out):
    return pl.pallas_call(
        functools.partial(_prep_from_acc, d_prev=d_prev, d_out=d_out),
        out_shape=[
            jax.ShapeDtypeStruct((N, DPAD), _f32),
            jax.ShapeDtypeStruct((N, 1), _f32),
            jax.ShapeDtypeStruct((N, 1), _f32),
            jax.ShapeDtypeStruct((N, 1), _f32),
        ],
    )(acc2, bp, wp, asp, adp)


def _tc_prep_final(acc2, bp, wap, wbp, bep, d_prev):
    return pl.pallas_call(
        functools.partial(_prep_final, d_prev=d_prev),
        out_shape=[
            jax.ShapeDtypeStruct((N, DPAD), _f32),
            jax.ShapeDtypeStruct((N, DPAD), _f32),
        ],
    )(acc2, bp, wap, wbp, bep)


def _make_gat_edge_kernel():
    mesh = plsc.VectorSubcoreMesh(core_axis_name="c", subcore_axis_name="s")

    @functools.partial(
        pl.kernel, mesh=mesh,
        compiler_params=pltpu.CompilerParams(needs_layout_passes=False, use_tc_tiling_on_sc=False),
        out_type=jax.ShapeDtypeStruct((2 * NN, DPAD), _f32),
        scratch_types=[
            pltpu.VMEM((NN,), _f32),
            pltpu.VMEM((NN,), _f32),
            pltpu.VMEM((NN,), _f32),
            pltpu.VMEM((NCH2, 128), jnp.int32),
            pltpu.VMEM((NCH2, 128), jnp.int32),
            pltpu.VMEM((128, DPAD), _f32),
            pltpu.VMEM((NPT, DPAD), _f32),
            pltpu.VMEM_SHARED((NN, DPAD), _f32),
            pltpu.VMEM_SHARED((NN, DPAD), _f32),
            pltpu.SemaphoreType.DMA,
        ],
    )
    def k(hp_hbm, hs_hbm, hd_hbm, c_hbm, s3_hbm, d3_hbm, out_hbm,
          hs_v, hd_v, c_v, s3v, d3v, rows_v, z_v, hsp, accsp, sem):
        core = lax.axis_index("c")
        sub = lax.axis_index("s")
        wid = sub * 2 + core
        i16 = lax.iota(jnp.int32, 16)
        zero16 = jnp.zeros((16,), _f32)

        pltpu.sync_copy(hs_hbm, hs_v)
        pltpu.sync_copy(hd_hbm, hd_v)
        pltpu.sync_copy(c_hbm, c_v)
        pltpu.sync_copy(s3_hbm.at[wid], s3v)
        pltpu.sync_copy(d3_hbm.at[wid], d3v)
        nslice = pl.ds(sub * NPT, NPT)
        pltpu.sync_copy(hp_hbm.at[nslice], hsp.at[nslice])
        for r in range(NPT):
            z_v[r, :] = zero16
        pltpu.sync_copy(z_v, accsp.at[nslice])
        plsc.subcore_barrier()

        ebase = wid * EPT2

        def chunk_body(j, carry):
            pltpu.async_copy(hsp.at[s3v.at[j]], rows_v, sem).wait()
            for g in range(8):
                off = j * 128 + g * 16
                s16 = s3v[j, pl.ds(g * 16, 16)]
                d16 = d3v[j, pl.ds(g * 16, 16)]
                hs_g = plsc.load_gather(hs_v, [s16])
                hd_g = plsc.load_gather(hd_v, [d16])
                c_g = plsc.load_gather(c_v, [d16])
                z = hs_g + hd_g
                lg = jnp.maximum(z, 0.2 * z)
                ex = jnp.exp(lg - c_g)
                eid = ebase + off + i16
                ex = jnp.where(eid < E2, ex, 0.0)
                for kk in range(16):
                    r = g * 16 + kk
                    exk = jnp.broadcast_to(ex[kk], (16,))
                    rows_v[r, :] = rows_v[r, :] * exk
            pltpu.sync_copy(rows_v, accsp.at[d3v.at[j]], add=True)
            return carry

        lax.fori_loop(0, NCH2, chunk_body, 0)
        plsc.subcore_barrier()
        pltpu.sync_copy(accsp.at[nslice],
                        out_hbm.at[pl.ds(core * NN + sub * NPT, NPT)])

    return k


def _make_edge_head_kernel():
    mesh = plsc.VectorSubcoreMesh(core_axis_name="c", subcore_axis_name="s")

    @functools.partial(
        pl.kernel, mesh=mesh,
        compiler_params=pltpu.CompilerParams(needs_layout_passes=False, use_tc_tiling_on_sc=False),
        out_type=jax.ShapeDtypeStruct((EPAD * 4,), _f32),
        scratch_types=[
            pltpu.VMEM((NCH, 128), jnp.int32),
            pltpu.VMEM((NCH, 128), jnp.int32),
            pltpu.VMEM((128, DPAD), _f32),
            pltpu.VMEM((128, DPAD), _f32),
            pltpu.VMEM((2048,), _f32),
            pltpu.VMEM((64,), _f32),
            pltpu.VMEM((16,), _f32),
            pltpu.VMEM((EPT * 4,), _f32),
            pltpu.VMEM_SHARED((NN, DPAD), _f32),
            pltpu.VMEM_SHARED((NN, DPAD), _f32),
            pltpu.SemaphoreType.DMA,
        ],
    )
    def k(p_hbm, q_hbm, s3_hbm, d3_hbm, w9_hbm, b9_hbm, out_hbm,
          s3v, d3v, pr_v, qr_v, ut_v, w9_v, b9_v, ob_v, psp, qsp, sem):
        core = lax.axis_index("c")
        sub = lax.axis_index("s")
        wid = sub * 2 + core
        i16 = lax.iota(jnp.int32, 16)

        pltpu.sync_copy(s3_hbm.at[wid], s3v)
        pltpu.sync_copy(d3_hbm.at[wid], d3v)
        pltpu.sync_copy(w9_hbm, w9_v)
        pltpu.sync_copy(b9_hbm, b9_v)
        nslice = pl.ds(sub * NPT, NPT)
        pltpu.sync_copy(p_hbm.at[nslice], psp.at[nslice])
        pltpu.sync_copy(q_hbm.at[nslice], qsp.at[nslice])
        plsc.subcore_barrier()

        w9cols = [w9_v[pl.ds(c * 16, 16)] for c in range(4)]
        b9all = b9_v[...]
        w9b = [[jnp.broadcast_to(w9cols[c][f], (16,)) for c in range(4)]
               for f in range(10)]
        b9b = [jnp.broadcast_to(b9all[c], (16,)) for c in range(4)]
        i16x128 = i16 * 128

        def chunk_body(j, carry):
            pltpu.async_copy(psp.at[d3v.at[j]], pr_v, sem).wait()
            pltpu.async_copy(qsp.at[s3v.at[j]], qr_v, sem).wait()
            for r in range(128):
                u = jnp.maximum(pr_v[r, :] + qr_v[r, :], 0.0)
                plsc.store_scatter(ut_v, [i16x128 + r], u)
            for g in range(8):
                o = [b9b[c] for c in range(4)]
                for f in range(10):
                    uf = ut_v[pl.ds(f * 128 + g * 16, 16)]
                    o = [o[c] + uf * w9b[f][c] for c in range(4)]
                o = [jnp.maximum(oc, 0.0) for oc in o]
                m = jnp.maximum(jnp.maximum(o[0], o[1]), jnp.maximum(o[2], o[3]))
                ev = [jnp.exp(oc - m) for oc in o]
                r = 1.0 / (ev[0] + ev[1] + ev[2] + ev[3])
                le4 = (j * 128 + g * 16) * 4 + i16 * 4
                for c in range(4):
                    plsc.store_scatter(ob_v, [le4 + c], ev[c] * r)
            return carry

        lax.fori_loop(0, NCH, chunk_body, 0)
        pltpu.sync_copy(ob_v, out_hbm.at[pl.ds(wid * EPT * 4, EPT * 4)])

    return k


_gat = _make_gat_edge_kernel()
_head = _make_edge_head_kernel()


def _padw(w, r, c):
    return jnp.zeros((r, c), _f32).at[:w.shape[0], :w.shape[1]].set(w)


def _padv(v, r):
    return jnp.zeros((r,), _f32).at[:v.shape[0]].set(v)


def _node_pad(a):
    if a.shape[1] == 1:
        return jnp.zeros((NN,), _f32).at[:N].set(a[:, 0])
    return jnp.zeros((NN, DPAD), _f32).at[:N].set(a)


def kernel(x, e, W1, a1s, a1d, b1, W2, a2s, a2d, b2, W3, a3s, a3d, b3, We, be,
           W9, b9, edge_index):
    si = jnp.arange(N, dtype=jnp.int32)
    s2_full = jnp.zeros((EPAD2,), jnp.int32).at[:E].set(edge_index[0]).at[E:E2].set(si)
    d2_full = jnp.zeros((EPAD2,), jnp.int32).at[:E].set(edge_index[1]).at[E:E2].set(si)
    sa3 = s2_full.reshape(NW, NCH2, 128)
    da3 = d2_full.reshape(NW, NCH2, 128)

    s_flat = jnp.zeros((EPAD,), jnp.int32).at[:E].set(edge_index[0])
    d_flat = jnp.zeros((EPAD,), jnp.int32).at[:E].set(edge_index[1])
    s3 = s_flat.reshape(NW, NCH, 128)
    d3 = d_flat.reshape(NW, NCH, 128)

    hp, hs, hd, c = _tc_prep_x(
        x, _padw(W1, 128, DPAD), _padw(a1s[:, None], DPAD, 1),
        _padw(a1d[:, None], DPAD, 1), 5)
    acc = _gat(_node_pad(hp), _node_pad(hs), _node_pad(hd), _node_pad(c),
               sa3, da3)
    acc2 = jnp.concatenate([acc[:N], acc[NN:NN + N]], axis=0)

    hp, hs, hd, c = _tc_prep_acc(
        acc2, _padv(b1, DPAD)[None, :], _padw(W2, DPAD, DPAD),
        _padw(a2s[:, None], DPAD, 1), _padw(a2d[:, None], DPAD, 1), 5, 10)
    acc = _gat(_node_pad(hp), _node_pad(hs), _node_pad(hd), _node_pad(c),
               sa3, da3)
    acc2 = jnp.concatenate([acc[:N], acc[NN:NN + N]], axis=0)

    hp, hs, hd, c = _tc_prep_acc(
        acc2, _padv(b2, DPAD)[None, :], _padw(W3, DPAD, DPAD),
        _padw(a3s[:, None], DPAD, 1), _padw(a3d[:, None], DPAD, 1), 10, 10)
    acc = _gat(_node_pad(hp), _node_pad(hs), _node_pad(hd), _node_pad(c),
               sa3, da3)
    acc2 = jnp.concatenate([acc[:N], acc[NN:NN + N]], axis=0)

    p, q = _tc_prep_final(
        acc2, _padv(b3, DPAD)[None, :], _padw(We[:10] - We[10:], DPAD, DPAD),
        _padw(We[10:], DPAD, DPAD), _padv(be, DPAD)[None, :], 10)
    w9cols = jnp.zeros((64,), _f32).at[0:10].set(W9[:, 0]).at[16:26].set(
        W9[:, 1]).at[32:42].set(W9[:, 2]).at[48:58].set(W9[:, 3])
    out = _head(_node_pad(p), _node_pad(q), s3, d3,
                w9cols, _padv(b9, 16))
    return out.reshape(EPAD, 4)[:E]
